# baseline (speedup 1.0000x reference)
import functools

import jax
import jax.numpy as jnp
from jax import lax
from jax.experimental import pallas as pl
from jax.experimental.pallas import tpu as pltpu
from jax.experimental.pallas import tpu_sc as plsc

N = 10000
D = 128
E = 320000
NC = 2
NS = 16
NW = NC * NS
C = 80
K = E // NW // C

HR = 5000
HP = 5120
SP = HP // NS
DUMMY = HP - 1


def _sc_mesh():
    return plsc.VectorSubcoreMesh(
        core_axis_name="c", subcore_axis_name="s", num_cores=NC, num_subcores=NS
    )


@functools.partial(
    pl.kernel,
    out_type=jax.ShapeDtypeStruct((NC, HP, D), jnp.float32),
    mesh=_sc_mesh(),
    scratch_types=[
        pltpu.VMEM((K, C), jnp.int32),
        pltpu.VMEM((K, C), jnp.int32),
        pltpu.VMEM((C, D), jnp.float32),
        pltpu.VMEM_SHARED((HP, D), jnp.float32),
        pltpu.SemaphoreType.DMA,
    ],
)
def _agg_kernel(y_hbm, src_hbm, dst_hbm, zeros_hbm, out_hbm,
                src_v, dst_v, rows_v, agg_sh, sem):
    c = lax.axis_index("c")
    s = lax.axis_index("s")
    wid = s * NC + c
    pltpu.sync_copy(src_hbm.at[wid], src_v)
    pltpu.sync_copy(dst_hbm.at[wid], dst_v)
    base = s * SP
    pltpu.sync_copy(zeros_hbm, agg_sh.at[pl.ds(base, SP)])
    plsc.subcore_barrier()

    def body(j, carry):
        pltpu.async_copy(y_hbm.at[src_v.at[j]], rows_v, sem).wait()
        pltpu.sync_copy(rows_v, agg_sh.at[dst_v.at[j]], add=True)
        return carry

    lax.fori_loop(0, K, body, 0)
    plsc.subcore_barrier()
    pltpu.sync_copy(agg_sh.at[pl.ds(base, SP)], out_hbm.at[c].at[pl.ds(base, SP)])


_BLK = 1000
_GRID = (N // _BLK,)
_NLO = HR // _BLK

_row_spec = pl.BlockSpec((_BLK, D), lambda i: (i, 0))
_lo_spec = pl.BlockSpec((NC, _BLK, D), lambda i: (0, jnp.minimum(i, _NLO - 1), 0))
_hi_spec = pl.BlockSpec((NC, _BLK, D), lambda i: (0, jnp.maximum(i - _NLO, 0), 0))
_b_spec = pl.BlockSpec((1, D), lambda i: (0, 0))
_w_spec = pl.BlockSpec((D, D), lambda i: (0, 0))


def _sel(lo_ref, hi_ref):
    i = pl.program_id(0)
    return jnp.where(i < _NLO, lo_ref[0] + lo_ref[1], hi_ref[0] + hi_ref[1])


def _dinv(dlo_ref, dhi_ref):
    return lax.rsqrt(_sel(dlo_ref, dhi_ref)[:, 0:1] + 1.0)


def _mm1_body(dlo_ref, dhi_ref, x_ref, w_ref, y_ref):
    xw = jnp.dot(x_ref[...], w_ref[...],
                 preferred_element_type=jnp.float32,
                 precision=lax.Precision.HIGHEST)
    y_ref[...] = xw * _dinv(dlo_ref, dhi_ref)


_mm1 = pl.pallas_call(
    _mm1_body,
    grid=_GRID,
    in_specs=[_lo_spec, _hi_spec, _row_spec, _w_spec],
    out_specs=_row_spec,
    out_shape=jax.ShapeDtypeStruct((N, D), jnp.float32),
)


def _mid_body(dlo_ref, dhi_ref, alo_ref, ahi_ref, y1_ref, b1_ref, w2_ref,
              y2_ref):
    dinv = _dinv(dlo_ref, dhi_ref)
    a = _sel(alo_ref, ahi_ref) + y1_ref[...]
    z = jnp.maximum(a * dinv + b1_ref[...], 0.0)
    y2_ref[...] = jnp.dot(z, w2_ref[...],
                          preferred_element_type=jnp.float32,
                          precision=lax.Precision.HIGHEST) * dinv


_mid = pl.pallas_call(
    _mid_body,
    grid=_GRID,
    in_specs=[_lo_spec, _hi_spec, _lo_spec, _hi_spec, _row_spec, _b_spec,
              _w_spec],
    out_specs=_row_spec,
    out_shape=jax.ShapeDtypeStruct((N, D), jnp.float32),
)


def _fin_body(dlo_ref, dhi_ref, alo_ref, ahi_ref, y2_ref, b2_ref, out_ref):
    a = _sel(alo_ref, ahi_ref) + y2_ref[...]
    out_ref[...] = a * _dinv(dlo_ref, dhi_ref) + b2_ref[...]


_fin = pl.pallas_call(
    _fin_body,
    grid=_GRID,
    in_specs=[_lo_spec, _hi_spec, _lo_spec, _hi_spec, _row_spec, _b_spec],
    out_specs=_row_spec,
    out_shape=jax.ShapeDtypeStruct((N, D), jnp.float32),
)


def kernel(x, edge_index, W1, b1, W2, b2):
    src = edge_index[0]
    dst = edge_index[1]
    src3 = src.reshape(NW, K, C)
    dlo3 = jnp.where(dst < HR, dst, DUMMY).reshape(NW, K, C)
    dhi3 = jnp.where(dst >= HR, dst - HR, DUMMY).reshape(NW, K, C)
    zero3 = jnp.zeros((NW, K, C), jnp.int32)
    ones_tab = jnp.ones((N, D), jnp.float32)
    zerosSP = jnp.zeros((SP, D), jnp.float32)
    b1r = b1.reshape(1, D)
    b2r = b2.reshape(1, D)

    deglo = _agg_kernel(ones_tab, zero3, dlo3, zerosSP)
    deghi = _agg_kernel(ones_tab, zero3, dhi3, zerosSP)
    y1 = _mm1(deglo, deghi, x, W1)
    agg1lo = _agg_kernel(y1, src3, dlo3, zerosSP)
    agg1hi = _agg_kernel(y1, src3, dhi3, zerosSP)
    y2 = _mid(deglo, deghi, agg1lo, agg1hi, y1, b1r, W2)
    agg2lo = _agg_kernel(y2, src3, dlo3, zerosSP)
    agg2hi = _agg_kernel(y2, src3, dhi3, zerosSP)
    return _fin(deglo, deghi, agg2lo, agg2hi, y2, b2r)


# speedup vs baseline: 1.0116x; 1.0116x over previous
import functools

import jax
import jax.numpy as jnp
from jax import lax
from jax.experimental import pallas as pl
from jax.experimental.pallas import tpu as pltpu
from jax.experimental.pallas import tpu_sc as plsc

N = 10000
D = 128
E = 320000
NC = 2
NS = 16
NW = NC * NS
C = 80
K = E // NW // C

HR = 5000
HP = 5120
SP = HP // NS
DUMMY = HP - 1


def _sc_mesh():
    return plsc.VectorSubcoreMesh(
        core_axis_name="c", subcore_axis_name="s", num_cores=NC, num_subcores=NS
    )


NB = 5


@functools.partial(
    pl.kernel,
    out_type=jax.ShapeDtypeStruct((NC, HP, D), jnp.float32),
    mesh=_sc_mesh(),
    scratch_types=[
        pltpu.VMEM((K, C), jnp.int32),
        pltpu.VMEM((K, C), jnp.int32),
        pltpu.VMEM((NB, C, D), jnp.float32),
        pltpu.VMEM_SHARED((HP, D), jnp.float32),
    ] + [pltpu.SemaphoreType.DMA] * NB,
)
def _agg_kernel(y_hbm, src_hbm, dst_hbm, zeros_hbm, out_hbm,
                src_v, dst_v, rows_v, agg_sh, *gsems):
    c = lax.axis_index("c")
    s = lax.axis_index("s")
    wid = s * NC + c
    pltpu.sync_copy(src_hbm.at[wid], src_v)
    pltpu.sync_copy(dst_hbm.at[wid], dst_v)
    base = s * SP
    pltpu.sync_copy(zeros_hbm, agg_sh.at[pl.ds(base, SP)])
    plsc.subcore_barrier()

    for b in range(NB):
        pltpu.async_copy(y_hbm.at[src_v.at[b]], rows_v.at[b], gsems[b])

    def group(g, carry):
        for b in range(NB):
            j = g * NB + b
            pltpu.make_async_copy(y_hbm.at[src_v.at[j]], rows_v.at[b],
                                  gsems[b]).wait()
            pltpu.sync_copy(rows_v.at[b], agg_sh.at[dst_v.at[j]], add=True)

            @pl.when(j + NB < K)
            def _():
                pltpu.async_copy(y_hbm.at[src_v.at[j + NB]], rows_v.at[b],
                                 gsems[b])
        return carry

    lax.fori_loop(0, K // NB, group, 0)
    plsc.subcore_barrier()
    pltpu.sync_copy(agg_sh.at[pl.ds(base, SP)], out_hbm.at[c].at[pl.ds(base, SP)])


_BLK = 1000
_GRID = (N // _BLK,)
_NLO = HR // _BLK

_row_spec = pl.BlockSpec((_BLK, D), lambda i: (i, 0))
_lo_spec = pl.BlockSpec((NC, _BLK, D), lambda i: (0, jnp.minimum(i, _NLO - 1), 0))
_hi_spec = pl.BlockSpec((NC, _BLK, D), lambda i: (0, jnp.maximum(i - _NLO, 0), 0))
_b_spec = pl.BlockSpec((1, D), lambda i: (0, 0))
_w_spec = pl.BlockSpec((D, D), lambda i: (0, 0))


def _sel(lo_ref, hi_ref):
    i = pl.program_id(0)
    return jnp.where(i < _NLO, lo_ref[0] + lo_ref[1], hi_ref[0] + hi_ref[1])


def _dinv(dlo_ref, dhi_ref):
    return lax.rsqrt(_sel(dlo_ref, dhi_ref)[:, 0:1] + 1.0)


def _mm1_body(dlo_ref, dhi_ref, x_ref, w_ref, y_ref):
    xw = jnp.dot(x_ref[...], w_ref[...],
                 preferred_element_type=jnp.float32,
                 precision=lax.Precision.HIGHEST)
    y_ref[...] = xw * _dinv(dlo_ref, dhi_ref)


_mm1 = pl.pallas_call(
    _mm1_body,
    grid=_GRID,
    in_specs=[_lo_spec, _hi_spec, _row_spec, _w_spec],
    out_specs=_row_spec,
    out_shape=jax.ShapeDtypeStruct((N, D), jnp.float32),
)


def _mid_body(dlo_ref, dhi_ref, alo_ref, ahi_ref, y1_ref, b1_ref, w2_ref,
              y2_ref):
    dinv = _dinv(dlo_ref, dhi_ref)
    a = _sel(alo_ref, ahi_ref) + y1_ref[...]
    z = jnp.maximum(a * dinv + b1_ref[...], 0.0)
    y2_ref[...] = jnp.dot(z, w2_ref[...],
                          preferred_element_type=jnp.float32,
                          precision=lax.Precision.HIGHEST) * dinv


_mid = pl.pallas_call(
    _mid_body,
    grid=_GRID,
    in_specs=[_lo_spec, _hi_spec, _lo_spec, _hi_spec, _row_spec, _b_spec,
              _w_spec],
    out_specs=_row_spec,
    out_shape=jax.ShapeDtypeStruct((N, D), jnp.float32),
)


def _fin_body(dlo_ref, dhi_ref, alo_ref, ahi_ref, y2_ref, b2_ref, out_ref):
    a = _sel(alo_ref, ahi_ref) + y2_ref[...]
    out_ref[...] = a * _dinv(dlo_ref, dhi_ref) + b2_ref[...]


_fin = pl.pallas_call(
    _fin_body,
    grid=_GRID,
    in_specs=[_lo_spec, _hi_spec, _lo_spec, _hi_spec, _row_spec, _b_spec],
    out_specs=_row_spec,
    out_shape=jax.ShapeDtypeStruct((N, D), jnp.float32),
)


def kernel(x, edge_index, W1, b1, W2, b2):
    src = edge_index[0]
    dst = edge_index[1]
    src3 = src.reshape(NW, K, C)
    dlo3 = jnp.where(dst < HR, dst, DUMMY).reshape(NW, K, C)
    dhi3 = jnp.where(dst >= HR, dst - HR, DUMMY).reshape(NW, K, C)
    zero3 = jnp.zeros((NW, K, C), jnp.int32)
    ones_tab = jnp.ones((N, D), jnp.float32)
    zerosSP = jnp.zeros((SP, D), jnp.float32)
    b1r = b1.reshape(1, D)
    b2r = b2.reshape(1, D)

    deglo = _agg_kernel(ones_tab, zero3, dlo3, zerosSP)
    deghi = _agg_kernel(ones_tab, zero3, dhi3, zerosSP)
    y1 = _mm1(deglo, deghi, x, W1)
    agg1lo = _agg_kernel(y1, src3, dlo3, zerosSP)
    agg1hi = _agg_kernel(y1, src3, dhi3, zerosSP)
    y2 = _mid(deglo, deghi, agg1lo, agg1hi, y1, b1r, W2)
    agg2lo = _agg_kernel(y2, src3, dlo3, zerosSP)
    agg2hi = _agg_kernel(y2, src3, dhi3, zerosSP)
    return _fin(deglo, deghi, agg2lo, agg2hi, y2, b2r)


# speedup vs baseline: 1.0164x; 1.0048x over previous
import functools

import jax
import jax.numpy as jnp
from jax import lax
from jax.experimental import pallas as pl
from jax.experimental.pallas import tpu as pltpu
from jax.experimental.pallas import tpu_sc as plsc

N = 10000
D = 128
E = 320000
NC = 2
NS = 16
NW = NC * NS
C = 80
K = E // NW // C

HR = 5000
HP = 5120
SP = HP // NS
DUMMY = HP - 1


def _sc_mesh():
    return plsc.VectorSubcoreMesh(
        core_axis_name="c", subcore_axis_name="s", num_cores=NC, num_subcores=NS
    )


NB = 5


@functools.partial(
    pl.kernel,
    out_type=jax.ShapeDtypeStruct((NC, HP, D), jnp.float32),
    mesh=_sc_mesh(),
    scratch_types=[
        pltpu.VMEM((K, C), jnp.int32),
        pltpu.VMEM((K, C), jnp.int32),
        pltpu.VMEM((NB, C, D), jnp.float32),
        pltpu.VMEM_SHARED((HP, D), jnp.float32),
    ] + [pltpu.SemaphoreType.DMA] * NB,
)
def _agg_kernel(y_hbm, src_hbm, dst_hbm, zeros_hbm, out_hbm,
                src_v, dst_v, rows_v, agg_sh, *gsems):
    c = lax.axis_index("c")
    s = lax.axis_index("s")
    wid = s * NC + c
    pltpu.sync_copy(src_hbm.at[wid], src_v)
    pltpu.sync_copy(dst_hbm.at[wid], dst_v)
    base = s * SP
    pltpu.sync_copy(zeros_hbm, agg_sh.at[pl.ds(base, SP)])
    plsc.subcore_barrier()

    for b in range(NB):
        pltpu.async_copy(y_hbm.at[src_v.at[b]], rows_v.at[b], gsems[b])

    def group(g, carry):
        for b in range(NB):
            j = g * NB + b
            pltpu.make_async_copy(y_hbm.at[src_v.at[j]], rows_v.at[b],
                                  gsems[b]).wait()
            pltpu.sync_copy(rows_v.at[b], agg_sh.at[dst_v.at[j]], add=True)

            @pl.when(j + NB < K)
            def _():
                pltpu.async_copy(y_hbm.at[src_v.at[j + NB]], rows_v.at[b],
                                 gsems[b])
        return carry

    lax.fori_loop(0, K // NB, group, 0)
    plsc.subcore_barrier()
    pltpu.sync_copy(agg_sh.at[pl.ds(base, SP)], out_hbm.at[c].at[pl.ds(base, SP)])


_BLK = 1000
_GRID = (N // _BLK,)
_NLO = HR // _BLK

_row_spec = pl.BlockSpec((_BLK, D), lambda i: (i, 0))
_lo_spec = pl.BlockSpec((NC, _BLK, D), lambda i: (0, jnp.minimum(i, _NLO - 1), 0))
_hi_spec = pl.BlockSpec((NC, _BLK, D), lambda i: (0, jnp.maximum(i - _NLO, 0), 0))
_b_spec = pl.BlockSpec((1, D), lambda i: (0, 0))
_w_spec = pl.BlockSpec((D, D), lambda i: (0, 0))


def _sel(lo_ref, hi_ref):
    i = pl.program_id(0)
    return jnp.where(i < _NLO, lo_ref[0] + lo_ref[1], hi_ref[0] + hi_ref[1])


def _dinv(dlo_ref, dhi_ref):
    return lax.rsqrt(_sel(dlo_ref, dhi_ref)[:, 0:1] + 1.0)


def _mm1_body(dlo_ref, dhi_ref, x_ref, w_ref, y_ref):
    xw = jnp.dot(x_ref[...], w_ref[...],
                 preferred_element_type=jnp.float32,
                 precision=lax.Precision.HIGHEST)
    y_ref[...] = xw * _dinv(dlo_ref, dhi_ref)


_mm1 = pl.pallas_call(
    _mm1_body,
    grid=_GRID,
    in_specs=[_lo_spec, _hi_spec, _row_spec, _w_spec],
    out_specs=_row_spec,
    out_shape=jax.ShapeDtypeStruct((N, D), jnp.float32),
)


def _mid_body(dlo_ref, dhi_ref, alo_ref, ahi_ref, y1_ref, b1_ref, w2_ref,
              y2_ref):
    dinv = _dinv(dlo_ref, dhi_ref)
    a = _sel(alo_ref, ahi_ref) + y1_ref[...]
    z = jnp.maximum(a * dinv + b1_ref[...], 0.0)
    y2_ref[...] = jnp.dot(z, w2_ref[...],
                          preferred_element_type=jnp.float32,
                          precision=lax.Precision.HIGHEST) * dinv


_mid = pl.pallas_call(
    _mid_body,
    grid=_GRID,
    in_specs=[_lo_spec, _hi_spec, _lo_spec, _hi_spec, _row_spec, _b_spec,
              _w_spec],
    out_specs=_row_spec,
    out_shape=jax.ShapeDtypeStruct((N, D), jnp.float32),
)


def _fin_body(dlo_ref, dhi_ref, alo_ref, ahi_ref, y2_ref, b2_ref, out_ref):
    a = _sel(alo_ref, ahi_ref) + y2_ref[...]
    out_ref[...] = a * _dinv(dlo_ref, dhi_ref) + b2_ref[...]


_fin = pl.pallas_call(
    _fin_body,
    grid=_GRID,
    in_specs=[_lo_spec, _hi_spec, _lo_spec, _hi_spec, _row_spec, _b_spec],
    out_specs=_row_spec,
    out_shape=jax.ShapeDtypeStruct((N, D), jnp.float32),
)


def kernel(x, edge_index, W1, b1, W2, b2):
    src = edge_index[0]
    dst = edge_index[1]
    src3 = src.reshape(NW, K, C)
    dmy = (HR + jnp.arange(NW, dtype=dst.dtype))[:, None]
    dst2 = dst.reshape(NW, K * C)
    dlo3 = jnp.where(dst2 < HR, dst2, dmy).reshape(NW, K, C)
    dhi3 = jnp.where(dst2 >= HR, dst2 - HR, dmy).reshape(NW, K, C)
    zero3 = jnp.zeros((NW, K, C), jnp.int32)
    ones_tab = jnp.ones((N, D), jnp.float32)
    zerosSP = jnp.zeros((SP, D), jnp.float32)
    b1r = b1.reshape(1, D)
    b2r = b2.reshape(1, D)

    deglo = _agg_kernel(ones_tab, zero3, dlo3, zerosSP)
    deghi = _agg_kernel(ones_tab, zero3, dhi3, zerosSP)
    y1 = _mm1(deglo, deghi, x, W1)
    agg1lo = _agg_kernel(y1, src3, dlo3, zerosSP)
    agg1hi = _agg_kernel(y1, src3, dhi3, zerosSP)
    y2 = _mid(deglo, deghi, agg1lo, agg1hi, y1, b1r, W2)
    agg2lo = _agg_kernel(y2, src3, dlo3, zerosSP)
    agg2hi = _agg_kernel(y2, src3, dhi3, zerosSP)
    return _fin(deglo, deghi, agg2lo, agg2hi, y2, b2r)
